# baseline (speedup 1.0000x reference)
import functools

import jax
import jax.numpy as jnp
from jax import lax
from jax.experimental import pallas as pl
from jax.experimental.pallas import tpu as pltpu
from jax.experimental.pallas import tpu_sc as plsc

_N = 10000
_E = 320000
_D = 128
_EPS = 1e-7

_ECH = 128
_NCHUNKS = _E // _ECH
_NTEC = 16
_BASE_CNT = _NCHUNKS // _NTEC
_REM = _NCHUNKS % _NTEC
_NODES_PER_TEC = _N // _NTEC
_NROW = 125


def _sc_message_body(h_hbm, ea_hbm, idx_hbm, t_hbm, out_hbm,
                     acc_s, idx0_v, idx1_v, hrow0_v, hrow1_v,
                     ea0_v, ea1_v, wp_v, t_v,
                     sem_i0, sem_i1, sem_g0, sem_g1, sem_e0, sem_e1):
    c = lax.axis_index("c")
    s = lax.axis_index("s")
    idxv = (idx0_v, idx1_v)
    hrowv = (hrow0_v, hrow1_v)
    eav = (ea0_v, ea1_v)
    sem_i = (sem_i0, sem_i1)
    sem_g = (sem_g0, sem_g1)
    sem_e = (sem_e0, sem_e1)

    def zrow(i, _):
        for k in range(8):
            wp_v[i, pl.ds(k * 16, 16)] = jnp.zeros((16,), jnp.float32)
        return 0
    lax.fori_loop(0, _NROW, zrow, 0)
    nb = s * _NODES_PER_TEC
    for q in range(_NODES_PER_TEC // _NROW):
        pltpu.sync_copy(wp_v.at[pl.ds(0, _NROW)],
                        acc_s.at[pl.ds(nb + q * _NROW, _NROW)])
    plsc.subcore_barrier()

    pltpu.sync_copy(t_hbm, t_v)
    tvec = t_v[...]

    cnt = _BASE_CNT + jnp.where(s < _REM, 1, 0)
    start = s * _BASE_CNT + jnp.minimum(s, _REM)
    cN = c * _N

    def ea_slice(g):
        eb = pl.multiple_of((start + g) * _ECH, 8)
        return ea_hbm.at[c, pl.ds(eb, _ECH)]

    def issue_pre(g, b):
        @pl.when(g < cnt)
        def _():
            pltpu.async_copy(idx_hbm.at[start + g], idxv[b], sem_i[b])
            pltpu.async_copy(ea_slice(g), eav[b], sem_e[b])

    def issue_gather(g, b):
        @pl.when(g < cnt)
        def _():
            pltpu.make_async_copy(idx_hbm.at[start + g], idxv[b],
                                  sem_i[b]).wait()
            for j in range(_ECH // 16):
                sl = pl.ds(j * 16, 16)
                idxv[b][0, sl] = idxv[b][0, sl] + cN
            pltpu.async_copy(h_hbm.at[idxv[b].at[0]], hrowv[b], sem_g[b])

    issue_pre(0, 0)
    issue_gather(0, 0)
    issue_pre(1, 1)

    def chunk_body(g, _):
        for b in range(2):
            @pl.when((g & 1) == b)
            def _():
                pltpu.make_async_copy(h_hbm.at[idxv[b].at[0]], hrowv[b],
                                      sem_g[b]).wait()
                pltpu.make_async_copy(ea_slice(g), eav[b], sem_e[b]).wait()
                issue_gather(g + 1, b ^ 1)

                hr, er = hrowv[b], eav[b]

                def edge_body(i, _):
                    for k in range(4):
                        sl = pl.ds(k * 16, 16)
                        m = jnp.maximum(hr[i, sl] + er[i, sl],
                                        0.0) + _EPS
                        w = jnp.exp(m * tvec)
                        wp_v[i, pl.ds(k * 16, 16)] = w
                        wp_v[i, pl.ds(64 + k * 16, 16)] = w * m
                    return 0
                lax.fori_loop(0, _ECH, edge_body, 0)

                pltpu.sync_copy(wp_v, acc_s.at[idxv[b].at[1]], add=True)
                issue_pre(g + 2, b)
        return 0
    lax.fori_loop(0, cnt, chunk_body, 0)
    plsc.subcore_barrier()

    for q in range(_NODES_PER_TEC // _NROW):
        row0 = nb + q * _NROW
        pltpu.sync_copy(acc_s.at[pl.ds(row0, _NROW)],
                        wp_v.at[pl.ds(0, _NROW)])

        def node_body(i, _):
            for k in range(4):
                hrow0_v[i, pl.ds(k * 16, 16)] = (
                    wp_v[i, pl.ds(64 + k * 16, 16)]
                    / (wp_v[i, pl.ds(k * 16, 16)] + 1e-16))
            return 0
        lax.fori_loop(0, _NROW, node_body, 0)
        pltpu.sync_copy(hrow0_v.at[pl.ds(0, _NROW)],
                        out_hbm.at[pl.ds(row0, _NROW), pl.ds(c * 64, 64)])


def _sc_message(h_split, ea_split, idx2, t_vec):
    mesh = plsc.VectorSubcoreMesh(core_axis_name="c", subcore_axis_name="s")
    fn = pl.kernel(
        _sc_message_body,
        mesh=mesh,
        compiler_params=pltpu.CompilerParams(use_tc_tiling_on_sc=False),
        out_type=jax.ShapeDtypeStruct((_N, _D), jnp.float32),
        scratch_types=[
            pltpu.VMEM_SHARED((_N, 128), jnp.float32),
            pltpu.VMEM((2, _ECH), jnp.int32),
            pltpu.VMEM((2, _ECH), jnp.int32),
            pltpu.VMEM((_ECH, 64), jnp.float32),
            pltpu.VMEM((_ECH, 64), jnp.float32),
            pltpu.VMEM((_ECH, 64), jnp.float32),
            pltpu.VMEM((_ECH, 64), jnp.float32),
            pltpu.VMEM((_ECH, 128), jnp.float32),
            pltpu.VMEM((16,), jnp.float32),
            pltpu.SemaphoreType.DMA,
            pltpu.SemaphoreType.DMA,
            pltpu.SemaphoreType.DMA,
            pltpu.SemaphoreType.DMA,
            pltpu.SemaphoreType.DMA,
            pltpu.SemaphoreType.DMA,
        ],
    )
    return fn(h_split, ea_split, idx2, t_vec)


def _ln(h, g, b):
    mu = jnp.mean(h, axis=-1, keepdims=True)
    d = h - mu
    var = jnp.mean(d * d, axis=-1, keepdims=True)
    return d / jnp.sqrt(var + 1e-5) * g + b


def _proj_split_only_body(x_ref, w_ref, b_ref, os_ref):
    y = jnp.dot(x_ref[...], w_ref[...],
                preferred_element_type=jnp.float32) + b_ref[...]
    os_ref[0] = y[:, :64]
    os_ref[1] = y[:, 64:]


def _tc_proj_split_only(x, W, b, rows, blk):
    return pl.pallas_call(
        _proj_split_only_body,
        grid=(rows // blk,),
        in_specs=[
            pl.BlockSpec((blk, _D), lambda i: (i, 0)),
            pl.BlockSpec((_D, _D), lambda i: (0, 0)),
            pl.BlockSpec((1, _D), lambda i: (0, 0)),
        ],
        out_specs=pl.BlockSpec((2, blk, 64), lambda i: (0, i, 0)),
        out_shape=jax.ShapeDtypeStruct((2, rows, 64), jnp.float32),
    )(x, W, b.reshape(1, _D))


def _proj_split_body(x_ref, w_ref, b_ref, o_ref, os_ref):
    y = jnp.dot(x_ref[...], w_ref[...],
                preferred_element_type=jnp.float32) + b_ref[...]
    o_ref[...] = y
    os_ref[0] = y[:, :64]
    os_ref[1] = y[:, 64:]


def _tc_proj_split(x, W, b, rows, blk):
    out, out_s = pl.pallas_call(
        _proj_split_body,
        grid=(rows // blk,),
        in_specs=[
            pl.BlockSpec((blk, _D), lambda i: (i, 0)),
            pl.BlockSpec((_D, _D), lambda i: (0, 0)),
            pl.BlockSpec((1, _D), lambda i: (0, 0)),
        ],
        out_specs=[
            pl.BlockSpec((blk, _D), lambda i: (i, 0)),
            pl.BlockSpec((2, blk, 64), lambda i: (0, i, 0)),
        ],
        out_shape=[
            jax.ShapeDtypeStruct((rows, _D), jnp.float32),
            jax.ShapeDtypeStruct((2, rows, 64), jnp.float32),
        ],
    )(x, W, b.reshape(1, _D))
    return out, out_s


def _node_body(aggr_ref, hin_ref, hprev_ref, w1_ref, b1_ref, g1_ref, be1_ref,
               w2_ref, b2_ref, gn_ref, bn_ref, hn_ref, hinf_ref, hins_ref):
    out = aggr_ref[...] + hin_ref[...]
    hh = jnp.dot(out, w1_ref[...],
                 preferred_element_type=jnp.float32) + b1_ref[...]
    hh = jax.nn.relu(_ln(hh, g1_ref[...], be1_ref[...]))
    hh = jnp.dot(hh, w2_ref[...],
                 preferred_element_type=jnp.float32) + b2_ref[...]
    hn = hprev_ref[...] + hh
    hn_ref[...] = hn
    act = jax.nn.relu(_ln(hn, gn_ref[...], bn_ref[...]))
    hinf_ref[...] = act
    hins_ref[0] = act[:, :64]
    hins_ref[1] = act[:, 64:]


def _tc_node(aggr, h_in, h_prev, W1, b1, g1, be1, W2, b2, g_next, b_next):
    blk = 2000
    spec_n = pl.BlockSpec((blk, _D), lambda i: (i, 0))
    spec_w1 = pl.BlockSpec((_D, 2 * _D), lambda i: (0, 0))
    spec_w2 = pl.BlockSpec((2 * _D, _D), lambda i: (0, 0))
    spec_v2 = pl.BlockSpec((1, 2 * _D), lambda i: (0, 0))
    spec_v1 = pl.BlockSpec((1, _D), lambda i: (0, 0))
    return pl.pallas_call(
        _node_body,
        grid=(_N // blk,),
        in_specs=[spec_n, spec_n, spec_n, spec_w1, spec_v2, spec_v2, spec_v2,
                  spec_w2, spec_v1, spec_v1, spec_v1],
        out_specs=[spec_n, spec_n,
                   pl.BlockSpec((2, blk, 64), lambda i: (0, i, 0))],
        out_shape=[
            jax.ShapeDtypeStruct((_N, _D), jnp.float32),
            jax.ShapeDtypeStruct((_N, _D), jnp.float32),
            jax.ShapeDtypeStruct((2, _N, 64), jnp.float32),
        ],
    )(aggr, h_in, h_prev, W1, b1.reshape(1, -1), g1.reshape(1, -1),
      be1.reshape(1, -1), W2, b2.reshape(1, -1), g_next.reshape(1, -1),
      b_next.reshape(1, -1))


def _final_body(aggr_ref, hin_ref, hprev_ref, w1_ref, b1_ref, g1_ref,
                be1_ref, w2_ref, b2_ref, g0_ref, b0_ref, wl_ref, bl_ref,
                y_ref):
    out = aggr_ref[...] + hin_ref[...]
    hh = jnp.dot(out, w1_ref[...],
                 preferred_element_type=jnp.float32) + b1_ref[...]
    hh = jax.nn.relu(_ln(hh, g1_ref[...], be1_ref[...]))
    hh = jnp.dot(hh, w2_ref[...],
                 preferred_element_type=jnp.float32) + b2_ref[...]
    hn = hprev_ref[...] + hh
    act = jax.nn.relu(_ln(hn, g0_ref[...], b0_ref[...]))
    y_ref[...] = jnp.dot(act, wl_ref[...],
                         preferred_element_type=jnp.float32) + bl_ref[...]


def _tc_final(aggr, h_in, h_prev, W1, b1, g1, be1, W2, b2, g0, b0,
              W_lin, b_lin):
    blk = 2000
    spec_n = pl.BlockSpec((blk, _D), lambda i: (i, 0))
    spec_w1 = pl.BlockSpec((_D, 2 * _D), lambda i: (0, 0))
    spec_w2 = pl.BlockSpec((2 * _D, _D), lambda i: (0, 0))
    spec_wl = pl.BlockSpec((_D, _D), lambda i: (0, 0))
    spec_v2 = pl.BlockSpec((1, 2 * _D), lambda i: (0, 0))
    spec_v1 = pl.BlockSpec((1, _D), lambda i: (0, 0))
    return pl.pallas_call(
        _final_body,
        grid=(_N // blk,),
        in_specs=[spec_n, spec_n, spec_n, spec_w1, spec_v2, spec_v2, spec_v2,
                  spec_w2, spec_v1, spec_v1, spec_v1, spec_wl, spec_v1],
        out_specs=spec_n,
        out_shape=jax.ShapeDtypeStruct((_N, _D), jnp.float32),
    )(aggr, h_in, h_prev, W1, b1.reshape(1, -1), g1.reshape(1, -1),
      be1.reshape(1, -1), W2, b2.reshape(1, -1), g0.reshape(1, -1),
      b0.reshape(1, -1), W_lin, b_lin.reshape(1, -1))


def kernel(x, edge_attr, edge_index, W_ne, b_ne, W_ee, b_ee, W1, b1, g_mlp,
           beta_mlp, W2, b2, t, g_ln, b_ln, W_lin, b_lin):
    idx2 = edge_index.reshape(2, _NCHUNKS, _ECH).transpose(1, 0, 2)

    ea_s = _tc_proj_split_only(edge_attr, W_ee, b_ee, _E, 2000)
    h_in, h_s = _tc_proj_split(x, W_ne, b_ne, _N, 2000)
    h_prev = jnp.zeros((_N, _D), jnp.float32)

    for i in range(4):
        t_vec = jnp.full((16,), t[i], jnp.float32)
        aggr = _sc_message(h_s.reshape(2 * _N, 64), ea_s, idx2, t_vec)
        if i < 3:
            h_prev, h_in, h_s = _tc_node(
                aggr, h_in, h_prev, W1[i], b1[i], g_mlp[i], beta_mlp[i],
                W2[i], b2[i], g_ln[i + 1], b_ln[i + 1])
        else:
            y = _tc_final(
                aggr, h_in, h_prev, W1[i], b1[i], g_mlp[i], beta_mlp[i],
                W2[i], b2[i], g_ln[0], b_ln[0], W_lin, b_lin)
    return y


# speedup vs baseline: 3.4652x; 3.4652x over previous
import functools

import jax
import jax.numpy as jnp
from jax import lax
from jax.experimental import pallas as pl
from jax.experimental.pallas import tpu as pltpu
from jax.experimental.pallas import tpu_sc as plsc

_N = 10000
_E = 320000
_D = 128
_EPS = 1e-7

_ECH = 128
_NCHUNKS = _E // _ECH
_NTEC = 16
_BASE_CNT = _NCHUNKS // _NTEC
_REM = _NCHUNKS % _NTEC
_NODES_PER_TEC = _N // _NTEC
_NROW = 125


def _sc_message_body(h_hbm, ea_hbm, idx_hbm, t_hbm, out_hbm,
                     num_s, den_s, idx0_v, idx1_v, hrow0_v, hrow1_v,
                     ea0_v, ea1_v, w_v, p_v, t_v,
                     sem_i0, sem_i1, sem_g0, sem_g1, sem_e0, sem_e1):
    c = lax.axis_index("c")
    s = lax.axis_index("s")
    idxv = (idx0_v, idx1_v)
    hrowv = (hrow0_v, hrow1_v)
    eav = (ea0_v, ea1_v)
    sem_i = (sem_i0, sem_i1)
    sem_g = (sem_g0, sem_g1)
    sem_e = (sem_e0, sem_e1)

    def zrow(i, _):
        for k in range(4):
            w_v[i, pl.ds(k * 16, 16)] = jnp.zeros((16,), jnp.float32)
        return 0
    lax.fori_loop(0, _NROW, zrow, 0)
    nb = s * _NODES_PER_TEC
    for q in range(_NODES_PER_TEC // _NROW):
        pltpu.sync_copy(w_v.at[pl.ds(0, _NROW)],
                        num_s.at[pl.ds(nb + q * _NROW, _NROW)])
        pltpu.sync_copy(w_v.at[pl.ds(0, _NROW)],
                        den_s.at[pl.ds(nb + q * _NROW, _NROW)])
    plsc.subcore_barrier()

    pltpu.sync_copy(t_hbm, t_v)
    tvec = t_v[...]

    cnt = _BASE_CNT + jnp.where(s < _REM, 1, 0)
    start = s * _BASE_CNT + jnp.minimum(s, _REM)
    cN = c * _N

    def ea_slice(g):
        eb = pl.multiple_of((start + g) * _ECH, 8)
        return ea_hbm.at[c, pl.ds(eb, _ECH)]

    def issue_pre(g, b):
        @pl.when(g < cnt)
        def _():
            pltpu.async_copy(idx_hbm.at[start + g], idxv[b], sem_i[b])
            pltpu.async_copy(ea_slice(g), eav[b], sem_e[b])

    def issue_gather(g, b):
        @pl.when(g < cnt)
        def _():
            pltpu.make_async_copy(idx_hbm.at[start + g], idxv[b],
                                  sem_i[b]).wait()
            for j in range(_ECH // 16):
                sl = pl.ds(j * 16, 16)
                idxv[b][0, sl] = idxv[b][0, sl] + cN
            pltpu.async_copy(h_hbm.at[idxv[b].at[0]], hrowv[b], sem_g[b])

    issue_pre(0, 0)
    issue_gather(0, 0)
    issue_pre(1, 1)

    def chunk_body(g, _):
        for b in range(2):
            @pl.when((g & 1) == b)
            def _():
                pltpu.make_async_copy(h_hbm.at[idxv[b].at[0]], hrowv[b],
                                      sem_g[b]).wait()
                pltpu.make_async_copy(ea_slice(g), eav[b], sem_e[b]).wait()
                issue_gather(g + 1, b ^ 1)

                hr, er = hrowv[b], eav[b]

                def edge_body(i, _):
                    for k in range(4):
                        sl = pl.ds(k * 16, 16)
                        m = jnp.maximum(hr[i, sl] + er[i, sl], 0.0) + _EPS
                        w = jnp.exp(m * tvec)
                        w_v[i, sl] = w
                        p_v[i, sl] = w * m
                    return 0
                lax.fori_loop(0, _ECH, edge_body, 0)

                pltpu.sync_copy(w_v, den_s.at[idxv[b].at[1]], add=True)
                pltpu.sync_copy(p_v, num_s.at[idxv[b].at[1]], add=True)
                issue_pre(g + 2, b)
        return 0
    lax.fori_loop(0, cnt, chunk_body, 0)
    plsc.subcore_barrier()

    for q in range(_NODES_PER_TEC // _NROW):
        row0 = nb + q * _NROW
        pltpu.sync_copy(num_s.at[pl.ds(row0, _NROW)],
                        hrow0_v.at[pl.ds(0, _NROW)])
        pltpu.sync_copy(den_s.at[pl.ds(row0, _NROW)],
                        ea0_v.at[pl.ds(0, _NROW)])

        def node_body(i, _):
            for k in range(4):
                sl = pl.ds(k * 16, 16)
                w_v[i, sl] = hrow0_v[i, sl] / (ea0_v[i, sl] + 1e-16)
            return 0
        lax.fori_loop(0, _NROW, node_body, 0)
        pltpu.sync_copy(w_v.at[pl.ds(0, _NROW)],
                        out_hbm.at[pl.ds(row0, _NROW), pl.ds(c * 64, 64)])


def _sc_message(h_split, ea_split, idx2, t_vec):
    mesh = plsc.VectorSubcoreMesh(core_axis_name="c", subcore_axis_name="s")
    fn = pl.kernel(
        _sc_message_body,
        mesh=mesh,
        compiler_params=pltpu.CompilerParams(use_tc_tiling_on_sc=False),
        out_type=jax.ShapeDtypeStruct((_N, _D), jnp.float32),
        scratch_types=[
            pltpu.VMEM_SHARED((_N, 64), jnp.float32),
            pltpu.VMEM_SHARED((_N, 64), jnp.float32),
            pltpu.VMEM((2, _ECH), jnp.int32),
            pltpu.VMEM((2, _ECH), jnp.int32),
            pltpu.VMEM((_ECH, 64), jnp.float32),
            pltpu.VMEM((_ECH, 64), jnp.float32),
            pltpu.VMEM((_ECH, 64), jnp.float32),
            pltpu.VMEM((_ECH, 64), jnp.float32),
            pltpu.VMEM((_ECH, 64), jnp.float32),
            pltpu.VMEM((_ECH, 64), jnp.float32),
            pltpu.VMEM((16,), jnp.float32),
            pltpu.SemaphoreType.DMA,
            pltpu.SemaphoreType.DMA,
            pltpu.SemaphoreType.DMA,
            pltpu.SemaphoreType.DMA,
            pltpu.SemaphoreType.DMA,
            pltpu.SemaphoreType.DMA,
        ],
    )
    return fn(h_split, ea_split, idx2, t_vec)


def _ln(h, g, b):
    mu = jnp.mean(h, axis=-1, keepdims=True)
    d = h - mu
    var = jnp.mean(d * d, axis=-1, keepdims=True)
    return d / jnp.sqrt(var + 1e-5) * g + b


def _proj_split_only_body(x_ref, w_ref, b_ref, os_ref):
    y = jnp.dot(x_ref[...], w_ref[...],
                preferred_element_type=jnp.float32) + b_ref[...]
    os_ref[0] = y[:, :64]
    os_ref[1] = y[:, 64:]


def _tc_proj_split_only(x, W, b, rows, blk):
    return pl.pallas_call(
        _proj_split_only_body,
        grid=(rows // blk,),
        in_specs=[
            pl.BlockSpec((blk, _D), lambda i: (i, 0)),
            pl.BlockSpec((_D, _D), lambda i: (0, 0)),
            pl.BlockSpec((1, _D), lambda i: (0, 0)),
        ],
        out_specs=pl.BlockSpec((2, blk, 64), lambda i: (0, i, 0)),
        out_shape=jax.ShapeDtypeStruct((2, rows, 64), jnp.float32),
    )(x, W, b.reshape(1, _D))


def _proj_split_body(x_ref, w_ref, b_ref, o_ref, os_ref):
    y = jnp.dot(x_ref[...], w_ref[...],
                preferred_element_type=jnp.float32) + b_ref[...]
    o_ref[...] = y
    os_ref[0] = y[:, :64]
    os_ref[1] = y[:, 64:]


def _tc_proj_split(x, W, b, rows, blk):
    out, out_s = pl.pallas_call(
        _proj_split_body,
        grid=(rows // blk,),
        in_specs=[
            pl.BlockSpec((blk, _D), lambda i: (i, 0)),
            pl.BlockSpec((_D, _D), lambda i: (0, 0)),
            pl.BlockSpec((1, _D), lambda i: (0, 0)),
        ],
        out_specs=[
            pl.BlockSpec((blk, _D), lambda i: (i, 0)),
            pl.BlockSpec((2, blk, 64), lambda i: (0, i, 0)),
        ],
        out_shape=[
            jax.ShapeDtypeStruct((rows, _D), jnp.float32),
            jax.ShapeDtypeStruct((2, rows, 64), jnp.float32),
        ],
    )(x, W, b.reshape(1, _D))
    return out, out_s


def _node_body(aggr_ref, hin_ref, hprev_ref, w1_ref, b1_ref, g1_ref, be1_ref,
               w2_ref, b2_ref, gn_ref, bn_ref, hn_ref, hinf_ref, hins_ref):
    out = aggr_ref[...] + hin_ref[...]
    hh = jnp.dot(out, w1_ref[...],
                 preferred_element_type=jnp.float32) + b1_ref[...]
    hh = jax.nn.relu(_ln(hh, g1_ref[...], be1_ref[...]))
    hh = jnp.dot(hh, w2_ref[...],
                 preferred_element_type=jnp.float32) + b2_ref[...]
    hn = hprev_ref[...] + hh
    hn_ref[...] = hn
    act = jax.nn.relu(_ln(hn, gn_ref[...], bn_ref[...]))
    hinf_ref[...] = act
    hins_ref[0] = act[:, :64]
    hins_ref[1] = act[:, 64:]


def _tc_node(aggr, h_in, h_prev, W1, b1, g1, be1, W2, b2, g_next, b_next):
    blk = 2000
    spec_n = pl.BlockSpec((blk, _D), lambda i: (i, 0))
    spec_w1 = pl.BlockSpec((_D, 2 * _D), lambda i: (0, 0))
    spec_w2 = pl.BlockSpec((2 * _D, _D), lambda i: (0, 0))
    spec_v2 = pl.BlockSpec((1, 2 * _D), lambda i: (0, 0))
    spec_v1 = pl.BlockSpec((1, _D), lambda i: (0, 0))
    return pl.pallas_call(
        _node_body,
        grid=(_N // blk,),
        in_specs=[spec_n, spec_n, spec_n, spec_w1, spec_v2, spec_v2, spec_v2,
                  spec_w2, spec_v1, spec_v1, spec_v1],
        out_specs=[spec_n, spec_n,
                   pl.BlockSpec((2, blk, 64), lambda i: (0, i, 0))],
        out_shape=[
            jax.ShapeDtypeStruct((_N, _D), jnp.float32),
            jax.ShapeDtypeStruct((_N, _D), jnp.float32),
            jax.ShapeDtypeStruct((2, _N, 64), jnp.float32),
        ],
    )(aggr, h_in, h_prev, W1, b1.reshape(1, -1), g1.reshape(1, -1),
      be1.reshape(1, -1), W2, b2.reshape(1, -1), g_next.reshape(1, -1),
      b_next.reshape(1, -1))


def _final_body(aggr_ref, hin_ref, hprev_ref, w1_ref, b1_ref, g1_ref,
                be1_ref, w2_ref, b2_ref, g0_ref, b0_ref, wl_ref, bl_ref,
                y_ref):
    out = aggr_ref[...] + hin_ref[...]
    hh = jnp.dot(out, w1_ref[...],
                 preferred_element_type=jnp.float32) + b1_ref[...]
    hh = jax.nn.relu(_ln(hh, g1_ref[...], be1_ref[...]))
    hh = jnp.dot(hh, w2_ref[...],
                 preferred_element_type=jnp.float32) + b2_ref[...]
    hn = hprev_ref[...] + hh
    act = jax.nn.relu(_ln(hn, g0_ref[...], b0_ref[...]))
    y_ref[...] = jnp.dot(act, wl_ref[...],
                         preferred_element_type=jnp.float32) + bl_ref[...]


def _tc_final(aggr, h_in, h_prev, W1, b1, g1, be1, W2, b2, g0, b0,
              W_lin, b_lin):
    blk = 2000
    spec_n = pl.BlockSpec((blk, _D), lambda i: (i, 0))
    spec_w1 = pl.BlockSpec((_D, 2 * _D), lambda i: (0, 0))
    spec_w2 = pl.BlockSpec((2 * _D, _D), lambda i: (0, 0))
    spec_wl = pl.BlockSpec((_D, _D), lambda i: (0, 0))
    spec_v2 = pl.BlockSpec((1, 2 * _D), lambda i: (0, 0))
    spec_v1 = pl.BlockSpec((1, _D), lambda i: (0, 0))
    return pl.pallas_call(
        _final_body,
        grid=(_N // blk,),
        in_specs=[spec_n, spec_n, spec_n, spec_w1, spec_v2, spec_v2, spec_v2,
                  spec_w2, spec_v1, spec_v1, spec_v1, spec_wl, spec_v1],
        out_specs=spec_n,
        out_shape=jax.ShapeDtypeStruct((_N, _D), jnp.float32),
    )(aggr, h_in, h_prev, W1, b1.reshape(1, -1), g1.reshape(1, -1),
      be1.reshape(1, -1), W2, b2.reshape(1, -1), g0.reshape(1, -1),
      b0.reshape(1, -1), W_lin, b_lin.reshape(1, -1))


def kernel(x, edge_attr, edge_index, W_ne, b_ne, W_ee, b_ee, W1, b1, g_mlp,
           beta_mlp, W2, b2, t, g_ln, b_ln, W_lin, b_lin):
    idx2 = edge_index.reshape(2, _NCHUNKS, _ECH).transpose(1, 0, 2)

    ea_s = _tc_proj_split_only(edge_attr, W_ee, b_ee, _E, 2000)
    h_in, h_s = _tc_proj_split(x, W_ne, b_ne, _N, 2000)
    h_prev = jnp.zeros((_N, _D), jnp.float32)

    for i in range(4):
        t_vec = jnp.full((16,), t[i], jnp.float32)
        aggr = _sc_message(h_s.reshape(2 * _N, 64), ea_s, idx2, t_vec)
        if i < 3:
            h_prev, h_in, h_s = _tc_node(
                aggr, h_in, h_prev, W1[i], b1[i], g_mlp[i], beta_mlp[i],
                W2[i], b2[i], g_ln[i + 1], b_ln[i + 1])
        else:
            y = _tc_final(
                aggr, h_in, h_prev, W1[i], b1[i], g_mlp[i], beta_mlp[i],
                W2[i], b2[i], g_ln[0], b_ln[0], W_lin, b_lin)
    return y
